# Initial kernel scaffold; baseline (speedup 1.0000x reference)
import functools

import jax
import jax.numpy as jnp
from jax import lax
from jax.experimental import pallas as pl
from jax.experimental.pallas import tpu as pltpu
from jax.experimental.pallas import tpu_sc as plsc

N_NODES = 10000
HID = 128
NUM_EDGES = 320000

NUM_SC = 2
NUM_SUB = 16
NUM_W = NUM_SC * NUM_SUB
EDGES_PER_W = NUM_EDGES // NUM_W
EDGE_BATCH = 80
NUM_BATCH = EDGES_PER_W // EDGE_BATCH
ROWS_PER_SUB = N_NODES // NUM_SUB


def _scatter_add_body(h_hbm, src_hbm, dst_hbm, zeros_hbm, out_hbm,
                      idx_s, idx_d, rows, agg_shared, sem):
    cid = lax.axis_index("c")
    sid = lax.axis_index("s")
    wid = sid * NUM_SC + cid

    row0 = sid * ROWS_PER_SUB
    pltpu.sync_copy(zeros_hbm.at[pl.ds(row0, ROWS_PER_SUB)],
                    agg_shared.at[pl.ds(row0, ROWS_PER_SUB)])
    plsc.subcore_barrier()

    base = wid * EDGES_PER_W

    def body(i, carry):
        off = base + i * EDGE_BATCH
        pltpu.sync_copy(src_hbm.at[pl.ds(off, EDGE_BATCH)], idx_s)
        pltpu.sync_copy(dst_hbm.at[pl.ds(off, EDGE_BATCH)], idx_d)
        pltpu.async_copy(h_hbm.at[idx_s], rows, sem).wait()
        pltpu.sync_copy(rows, agg_shared.at[idx_d], add=True)
        return carry

    lax.fori_loop(0, NUM_BATCH, body, 0)
    plsc.subcore_barrier()

    pltpu.sync_copy(agg_shared.at[pl.ds(row0, ROWS_PER_SUB)],
                    out_hbm.at[cid, pl.ds(row0, ROWS_PER_SUB)])


_scatter_add = functools.partial(
    pl.kernel,
    mesh=plsc.VectorSubcoreMesh(core_axis_name="c", subcore_axis_name="s"),
    out_type=jax.ShapeDtypeStruct((NUM_SC, N_NODES, HID), jnp.float32),
    scratch_types=[
        pltpu.VMEM((EDGE_BATCH,), jnp.int32),
        pltpu.VMEM((EDGE_BATCH,), jnp.int32),
        pltpu.VMEM((EDGE_BATCH, HID), jnp.float32),
        pltpu.VMEM_SHARED((N_NODES, HID), jnp.float32),
        pltpu.SemaphoreType.DMA,
    ],
)(_scatter_add_body)


ROW_BLK = 1000
GRID = N_NODES // ROW_BLK


def _gru_block(p0, p1, h, wih_t, whh_t, bih, bhh):
    agg = p0 + p1
    gi = jnp.dot(agg, wih_t, preferred_element_type=jnp.float32) + bih
    gh = jnp.dot(h, whh_t, preferred_element_type=jnp.float32) + bhh
    r = jax.nn.sigmoid(gi[:, :HID] + gh[:, :HID])
    z = jax.nn.sigmoid(gi[:, HID:2 * HID] + gh[:, HID:2 * HID])
    n = jnp.tanh(gi[:, 2 * HID:] + r * gh[:, 2 * HID:])
    return (1.0 - z) * n + z * h


def _gru_body(p_ref, h_ref, wih_ref, whh_ref, bih_ref, bhh_ref, out_ref):
    out_ref[...] = _gru_block(p_ref[0], p_ref[1], h_ref[...],
                              wih_ref[...], whh_ref[...],
                              bih_ref[...], bhh_ref[...])


def _gru_dense_max_body(p_ref, h_ref, wih_ref, whh_ref, bih_ref, bhh_ref,
                        wd_ref, bd_ref, out_ref):
    h_new = _gru_block(p_ref[0], p_ref[1], h_ref[...],
                       wih_ref[...], whh_ref[...],
                       bih_ref[...], bhh_ref[...])
    d = jnp.dot(h_new, wd_ref[...], preferred_element_type=jnp.float32) + bd_ref[...]
    blk_max = jnp.max(d, axis=0, keepdims=True)

    @pl.when(pl.program_id(0) == 0)
    def _():
        out_ref[...] = jnp.full_like(out_ref, -jnp.inf)

    out_ref[...] = jnp.maximum(out_ref[...], blk_max)


_row_spec = pl.BlockSpec((ROW_BLK, HID), lambda i: (i, 0))
_part_spec = pl.BlockSpec((NUM_SC, ROW_BLK, HID), lambda i: (0, i, 0))
_full = lambda shape: pl.BlockSpec(shape, lambda i: tuple(0 for _ in shape))

_gru_call = pl.pallas_call(
    _gru_body,
    grid=(GRID,),
    in_specs=[
        _part_spec,
        _row_spec,
        _full((HID, 3 * HID)),
        _full((HID, 3 * HID)),
        _full((1, 3 * HID)),
        _full((1, 3 * HID)),
    ],
    out_specs=_row_spec,
    out_shape=jax.ShapeDtypeStruct((N_NODES, HID), jnp.float32),
)

_gru_dense_max_call = pl.pallas_call(
    _gru_dense_max_body,
    grid=(GRID,),
    in_specs=[
        _part_spec,
        _row_spec,
        _full((HID, 3 * HID)),
        _full((HID, 3 * HID)),
        _full((1, 3 * HID)),
        _full((1, 3 * HID)),
        _full((HID, HID)),
        _full((1, HID)),
    ],
    out_specs=pl.BlockSpec((1, HID), lambda i: (0, 0)),
    out_shape=jax.ShapeDtypeStruct((1, HID), jnp.float32),
)


def kernel(x, edge_index,
           W_ih_0, W_hh_0, b_ih_0, b_hh_0,
           W_ih_1, W_hh_1, b_ih_1, b_hh_1,
           W_ih_2, W_hh_2, b_ih_2, b_hh_2,
           W_dense, b_dense):
    src = edge_index[0].astype(jnp.int32)
    dst = edge_index[1].astype(jnp.int32)
    zeros = jnp.zeros((N_NODES, HID), jnp.float32)

    params = [(W_ih_0, W_hh_0, b_ih_0, b_hh_0),
              (W_ih_1, W_hh_1, b_ih_1, b_hh_1),
              (W_ih_2, W_hh_2, b_ih_2, b_hh_2)]

    h = x
    for layer, (W_ih, W_hh, b_ih, b_hh) in enumerate(params):
        parts = _scatter_add(h, src, dst, zeros)
        args = (parts, h, W_ih.T, W_hh.T,
                b_ih.reshape(1, -1), b_hh.reshape(1, -1))
        if layer < 2:
            h = _gru_call(*args)
        else:
            out = _gru_dense_max_call(*args, W_dense.T,
                                      b_dense.reshape(1, -1))
    return out[0]


# speedup vs baseline: 4.4856x; 4.4856x over previous
import functools

import jax
import jax.numpy as jnp
from jax import lax
from jax.experimental import pallas as pl
from jax.experimental.pallas import tpu as pltpu
from jax.experimental.pallas import tpu_sc as plsc

N_NODES = 10000
HID = 128
NUM_EDGES = 320000

NUM_SC = 2
NUM_SUB = 16
NUM_W = NUM_SC * NUM_SUB
EDGES_PER_W = NUM_EDGES // NUM_W
EDGE_BATCH = 80
NUM_BATCH = EDGES_PER_W // EDGE_BATCH
N_PAD = 10240
ROWS_PER_SUB = N_PAD // NUM_SUB


def _scatter_add_body(h_hbm, src_hbm, dst_hbm, zeros_hbm, out_hbm,
                      idx_s, idx_d, rows, agg_shared, sem):
    cid = lax.axis_index("c")
    sid = lax.axis_index("s")
    wid = sid * NUM_SC + cid

    row0 = sid * ROWS_PER_SUB
    pltpu.sync_copy(zeros_hbm.at[pl.ds(row0, ROWS_PER_SUB)],
                    agg_shared.at[pl.ds(row0, ROWS_PER_SUB)])
    plsc.subcore_barrier()

    base = wid * EDGES_PER_W

    def body(i, carry):
        off = base + i * EDGE_BATCH
        pltpu.sync_copy(src_hbm.at[pl.ds(off, EDGE_BATCH)], idx_s)
        pltpu.sync_copy(dst_hbm.at[pl.ds(off, EDGE_BATCH)], idx_d)
        pltpu.async_copy(h_hbm.at[idx_s], rows, sem).wait()
        pltpu.sync_copy(rows, agg_shared.at[idx_d], add=True)
        return carry

    lax.fori_loop(0, NUM_BATCH, body, 0)
    plsc.subcore_barrier()

    pltpu.sync_copy(agg_shared.at[pl.ds(row0, ROWS_PER_SUB)],
                    out_hbm.at[cid, pl.ds(row0, ROWS_PER_SUB)])


@functools.cache
def _scatter_add():
    return pl.kernel(
        _scatter_add_body,
        mesh=plsc.VectorSubcoreMesh(core_axis_name="c", subcore_axis_name="s"),
        out_type=jax.ShapeDtypeStruct((NUM_SC, N_PAD, HID), jnp.float32),
        scratch_types=[
            pltpu.VMEM((EDGE_BATCH,), jnp.int32),
            pltpu.VMEM((EDGE_BATCH,), jnp.int32),
            pltpu.VMEM((EDGE_BATCH, HID), jnp.float32),
            pltpu.VMEM_SHARED((N_PAD, HID), jnp.float32),
            pltpu.SemaphoreType.DMA,
        ],
    )


ROW_BLK = 1000
GRID = N_NODES // ROW_BLK


def _gru_block(p0, p1, h, wih_t, whh_t, bih, bhh):
    agg = p0 + p1
    gi = jnp.dot(agg, wih_t, preferred_element_type=jnp.float32) + bih
    gh = jnp.dot(h, whh_t, preferred_element_type=jnp.float32) + bhh
    r = jax.nn.sigmoid(gi[:, :HID] + gh[:, :HID])
    z = jax.nn.sigmoid(gi[:, HID:2 * HID] + gh[:, HID:2 * HID])
    n = jnp.tanh(gi[:, 2 * HID:] + r * gh[:, 2 * HID:])
    return (1.0 - z) * n + z * h


def _gru_body(p_ref, h_ref, wih_ref, whh_ref, bih_ref, bhh_ref, out_ref):
    out_ref[...] = _gru_block(p_ref[0], p_ref[1], h_ref[...],
                              wih_ref[...], whh_ref[...],
                              bih_ref[...], bhh_ref[...])


def _gru_dense_max_body(p_ref, h_ref, wih_ref, whh_ref, bih_ref, bhh_ref,
                        wd_ref, bd_ref, out_ref):
    h_new = _gru_block(p_ref[0], p_ref[1], h_ref[...],
                       wih_ref[...], whh_ref[...],
                       bih_ref[...], bhh_ref[...])
    d = jnp.dot(h_new, wd_ref[...], preferred_element_type=jnp.float32) + bd_ref[...]
    blk_max = jnp.max(d, axis=0, keepdims=True)

    @pl.when(pl.program_id(0) == 0)
    def _():
        out_ref[...] = jnp.full_like(out_ref, -jnp.inf)

    out_ref[...] = jnp.maximum(out_ref[...], blk_max)


_row_spec = pl.BlockSpec((ROW_BLK, HID), lambda i: (i, 0))
_part_spec = pl.BlockSpec((NUM_SC, ROW_BLK, HID), lambda i: (0, i, 0))
_full = lambda shape: pl.BlockSpec(shape, lambda i: tuple(0 for _ in shape))

_gru_call = pl.pallas_call(
    _gru_body,
    grid=(GRID,),
    in_specs=[
        _part_spec,
        _row_spec,
        _full((HID, 3 * HID)),
        _full((HID, 3 * HID)),
        _full((1, 3 * HID)),
        _full((1, 3 * HID)),
    ],
    out_specs=_row_spec,
    out_shape=jax.ShapeDtypeStruct((N_NODES, HID), jnp.float32),
)

_gru_dense_max_call = pl.pallas_call(
    _gru_dense_max_body,
    grid=(GRID,),
    in_specs=[
        _part_spec,
        _row_spec,
        _full((HID, 3 * HID)),
        _full((HID, 3 * HID)),
        _full((1, 3 * HID)),
        _full((1, 3 * HID)),
        _full((HID, HID)),
        _full((1, HID)),
    ],
    out_specs=pl.BlockSpec((1, HID), lambda i: (0, 0)),
    out_shape=jax.ShapeDtypeStruct((1, HID), jnp.float32),
)


def kernel(x, edge_index,
           W_ih_0, W_hh_0, b_ih_0, b_hh_0,
           W_ih_1, W_hh_1, b_ih_1, b_hh_1,
           W_ih_2, W_hh_2, b_ih_2, b_hh_2,
           W_dense, b_dense):
    src = edge_index[0].astype(jnp.int32)
    dst = edge_index[1].astype(jnp.int32)
    zeros = jnp.zeros((N_PAD, HID), jnp.float32)

    params = [(W_ih_0, W_hh_0, b_ih_0, b_hh_0),
              (W_ih_1, W_hh_1, b_ih_1, b_hh_1),
              (W_ih_2, W_hh_2, b_ih_2, b_hh_2)]

    h = x
    for layer, (W_ih, W_hh, b_ih, b_hh) in enumerate(params):
        parts = _scatter_add()(h, src, dst, zeros)[:, :N_NODES]
        args = (parts, h, W_ih.T, W_hh.T,
                b_ih.reshape(1, -1), b_hh.reshape(1, -1))
        if layer < 2:
            h = _gru_call(*args)
        else:
            out = _gru_dense_max_call(*args, W_dense.T,
                                      b_dense.reshape(1, -1))
    return out[0]
